# Initial kernel scaffold; baseline (speedup 1.0000x reference)
import jax
import jax.numpy as jnp
from jax.experimental import pallas as pl


def kernel(x, tok_embed, pos_embed, gamma, beta):
    raise NotImplementedError("write your pallas kernel here")


# speedup vs baseline: 16.0738x; 16.0738x over previous
import functools

import jax
import jax.numpy as jnp
from jax import lax
from jax.experimental import pallas as pl
from jax.experimental.pallas import tpu as pltpu
from jax.experimental.pallas import tpu_sc as plsc

_NUM_CORES = 2
_NUM_SUBCORES = 16
_NW = _NUM_CORES * _NUM_SUBCORES

_EPS = 1e-5


def _table_body(tok_ref, pos_ref, g_ref, b_ref, out_ref):
    pos = pos_ref[...]
    g = g_ref[...]
    b = b_ref[...]
    vocab = tok_ref.shape[0]
    rows = []
    for v in range(vocab):
        e = pos + tok_ref[v:v + 1, :]
        m = jnp.mean(e, axis=1, keepdims=True)
        c = e - m
        var = jnp.mean(c * c, axis=1, keepdims=True)
        rows.append(c * lax.rsqrt(var + _EPS) * g + b)
    out_ref[...] = jnp.concatenate(rows, axis=0)


def _table_call(tok, pos, gamma, beta):
    vocab, d = tok.shape
    l = pos.shape[0]
    return pl.pallas_call(
        _table_body,
        out_shape=jax.ShapeDtypeStruct((vocab * l, d), jnp.float32),
    )(tok, pos, gamma.reshape(1, d), beta.reshape(1, d))


@functools.cache
def _gather_call(tok_total: int, l: int, d: int):
    chunk = 1024
    sub = 128
    nsub = chunk // sub
    per_w = tok_total // _NW
    nit = per_w // chunk
    assert per_w * _NW == tok_total and nit * chunk == per_w

    mesh = plsc.VectorSubcoreMesh(core_axis_name="c", subcore_axis_name="s")

    @functools.partial(
        pl.kernel,
        mesh=mesh,
        compiler_params=pltpu.CompilerParams(use_tc_tiling_on_sc=False),
        out_type=jax.ShapeDtypeStruct((tok_total, d), jnp.float32),
        scratch_types=[
            pltpu.VMEM((chunk,), jnp.int32),
            pltpu.VMEM((nsub, sub), jnp.int32),
            pltpu.VMEM((chunk, d), jnp.float32),
            pltpu.SemaphoreType.DMA,
        ],
    )
    def gather(table_hbm, xflat_hbm, out_hbm, x_v, idx_v, rows_v, sem):
        wid = lax.axis_index("s") * _NUM_CORES + lax.axis_index("c")
        lane = lax.iota(jnp.int32, 16)

        def body(it, carry):
            base = wid * per_w + it * chunk
            pltpu.sync_copy(xflat_hbm.at[pl.ds(base, chunk)], x_v)
            for j in range(nsub):
                for g16 in range(sub // 16):
                    off = j * sub + g16 * 16
                    p = (base + off) + lane
                    xv = x_v[pl.ds(off, 16)]
                    idx_v[j, pl.ds(g16 * 16, 16)] = xv * l + p % l
            copies = [
                pltpu.async_copy(
                    table_hbm.at[idx_v.at[j]],
                    rows_v.at[pl.ds(j * sub, sub)],
                    sem,
                )
                for j in range(nsub)
            ]
            for c in copies:
                c.wait()
            pltpu.sync_copy(rows_v, out_hbm.at[pl.ds(base, chunk)])
            return carry

        lax.fori_loop(0, nit, body, None)

    return gather


def kernel(x, tok_embed, pos_embed, gamma, beta):
    b, l = x.shape
    d = tok_embed.shape[1]
    table = _table_call(tok_embed, pos_embed, gamma, beta)
    out = _gather_call(b * l, l, d)(table, x.reshape(-1))
    return out.reshape(b, l, d)


# speedup vs baseline: 26.5774x; 1.6535x over previous
import functools

import jax
import jax.numpy as jnp
from jax import lax
from jax.experimental import pallas as pl
from jax.experimental.pallas import tpu as pltpu
from jax.experimental.pallas import tpu_sc as plsc

_NUM_CORES = 2
_NUM_SUBCORES = 16
_NW = _NUM_CORES * _NUM_SUBCORES

_EPS = 1e-5


def _table_body(tok_ref, pos_ref, g_ref, b_ref, out_ref):
    pos = pos_ref[...]
    g = g_ref[...]
    b = b_ref[...]
    vocab = tok_ref.shape[0]
    rows = []
    for v in range(vocab):
        e = pos + tok_ref[v:v + 1, :]
        m = jnp.mean(e, axis=1, keepdims=True)
        c = e - m
        var = jnp.mean(c * c, axis=1, keepdims=True)
        rows.append(c * lax.rsqrt(var + _EPS) * g + b)
    out_ref[...] = jnp.concatenate(rows, axis=0)


def _table_call(tok, pos, gamma, beta):
    vocab, d = tok.shape
    l = pos.shape[0]
    return pl.pallas_call(
        _table_body,
        out_shape=jax.ShapeDtypeStruct((vocab * l, d), jnp.float32),
    )(tok, pos, gamma.reshape(1, d), beta.reshape(1, d))


@functools.cache
def _gather_call(tok_total: int, l: int, d: int, tab_rows: int):
    chunk = 512
    sub = 128
    nsub = chunk // sub
    per_w = tok_total // _NW
    nit = per_w // chunk
    assert per_w * _NW == tok_total and nit * chunk == per_w
    assert nit % 2 == 0 and nit >= 8

    mesh = plsc.VectorSubcoreMesh(core_axis_name="c", subcore_axis_name="s")

    @functools.partial(
        pl.kernel,
        mesh=mesh,
        compiler_params=pltpu.CompilerParams(use_tc_tiling_on_sc=False),
        out_type=jax.ShapeDtypeStruct((tok_total, d), jnp.float32),
        scratch_types=[
            pltpu.VMEM_SHARED((tab_rows, d), jnp.float32),
            pltpu.VMEM((2, chunk), jnp.int32),
            pltpu.VMEM((2, nsub, sub), jnp.int32),
            pltpu.VMEM((2, chunk, d), jnp.float32),
            pltpu.SemaphoreType.DMA,
            [pltpu.SemaphoreType.DMA] * 2,
            [pltpu.SemaphoreType.DMA] * 2,
            [pltpu.SemaphoreType.DMA] * 2,
        ],
    )
    def gather(table_hbm, xflat_hbm, out_hbm, tab_s, x_v, idx_v, rows_v,
               tab_sem, x_sems, g_sems, o_sems):
        sid = lax.axis_index("s")
        wid = sid * _NUM_CORES + lax.axis_index("c")
        lane = lax.iota(jnp.int32, 16)
        w_base = wid * per_w

        @pl.when(sid == 0)
        def _():
            pltpu.async_copy(table_hbm, tab_s, tab_sem).wait()
        plsc.subcore_barrier()

        def x_fetch(it, b):
            return pltpu.async_copy(
                xflat_hbm.at[pl.ds(w_base + it * chunk, chunk)],
                x_v.at[b], x_sems[b])

        def compute_idx(it, b):
            base = w_base + it * chunk
            for j in range(nsub):
                for g16 in range(sub // 16):
                    off = j * sub + g16 * 16
                    p = (base + off) + lane
                    xv = x_v[b, pl.ds(off, 16)]
                    idx_v[b, j, pl.ds(g16 * 16, 16)] = xv * l + p % l

        def issue_gathers(b):
            return [
                pltpu.async_copy(
                    tab_s.at[idx_v.at[b].at[j]],
                    rows_v.at[b].at[pl.ds(j * sub, sub)],
                    g_sems[b])
                for j in range(nsub)
            ]

        def drain_gathers(b):
            for j in range(nsub):
                pltpu.make_async_copy(
                    tab_s.at[idx_v.at[b].at[j]],
                    rows_v.at[b].at[pl.ds(j * sub, sub)],
                    g_sems[b]).wait()

        def out_write(it, b):
            return pltpu.async_copy(
                rows_v.at[b], out_hbm.at[pl.ds(w_base + it * chunk, chunk)],
                o_sems[b])

        def drain_x(b):
            pltpu.make_async_copy(
                xflat_hbm.at[pl.ds(0, chunk)], x_v.at[b], x_sems[b]).wait()

        x_fetch(0, 0)
        x_fetch(1, 1)
        for b in (0, 1):
            drain_x(b)
            compute_idx(b, b)
            issue_gathers(b)
            x_fetch(b + 2, b)

        def body2(it2, carry):
            for b in (0, 1):
                it = it2 * 2 + b
                drain_gathers(b)
                ow = out_write(it, b)
                drain_x(b)
                compute_idx(it + 2, b)
                ow.wait()
                issue_gathers(b)
                x_fetch(it + 4, b)
            return carry

        lax.fori_loop(0, (nit - 4) // 2, body2, None, unroll=False)

        for k in range(4):
            it = nit - 4 + k
            b = it % 2
            drain_gathers(b)
            ow = out_write(it, b)
            if k < 2:
                drain_x(b)
                compute_idx(it + 2, b)
                ow.wait()
                issue_gathers(b)
            else:
                ow.wait()

    return gather


def kernel(x, tok_embed, pos_embed, gamma, beta):
    b, l = x.shape
    vocab, d = tok_embed.shape
    table = _table_call(tok_embed, pos_embed, gamma, beta)
    out = _gather_call(b * l, l, d, vocab * l)(table, x.reshape(-1))
    return out.reshape(b, l, d)


# speedup vs baseline: 91.5727x; 3.4455x over previous
import functools

import jax
import jax.numpy as jnp
from jax import lax
from jax.experimental import pallas as pl

_EPS = 1e-5


def _body(xt_ref, tok_ref, pos_ref, g_ref, b_ref, out_ref):
    pos = pos_ref[0]
    g = g_ref[...]
    bta = b_ref[...]
    vocab, d = tok_ref.shape
    xt = xt_ref[0]
    acc = None
    for v in range(vocab):
        e = pos + tok_ref[v:v + 1, :]
        m = jnp.mean(e, axis=1, keepdims=True)
        c = e - m
        var = jnp.mean(c * c, axis=1, keepdims=True)
        row = c * lax.rsqrt(var + _EPS) * g + bta
        col = row.reshape(d, 1)
        if acc is None:
            acc = jnp.broadcast_to(col, out_ref.shape[1:])
        else:
            acc = jnp.where(xt == v, col, acc)
    out_ref[...] = acc[None]


@functools.cache
def _call(nb: int, l: int, vocab: int, d: int, bb: int):
    grid = (l, nb // bb)
    return pl.pallas_call(
        _body,
        grid=grid,
        in_specs=[
            pl.BlockSpec((1, 1, bb), lambda i, j: (i, 0, j)),
            pl.BlockSpec((vocab, d), lambda i, j: (0, 0)),
            pl.BlockSpec((1, 1, d), lambda i, j: (i, 0, 0)),
            pl.BlockSpec((1, d), lambda i, j: (0, 0)),
            pl.BlockSpec((1, d), lambda i, j: (0, 0)),
        ],
        out_specs=pl.BlockSpec((1, d, bb), lambda i, j: (i, 0, j)),
        out_shape=jax.ShapeDtypeStruct((l, d, nb), jnp.float32),
    )


def kernel(x, tok_embed, pos_embed, gamma, beta):
    nb, l = x.shape
    vocab, d = tok_embed.shape
    out_t = _call(nb, l, vocab, d, 4096)(
        x.T.reshape(l, 1, nb), tok_embed, pos_embed.reshape(l, 1, d),
        gamma.reshape(1, d), beta.reshape(1, d))
    return jnp.transpose(out_t, (2, 0, 1))
